# baseline (speedup 1.0000x reference)
import functools

import jax
import jax.numpy as jnp
from jax import lax
from jax.experimental import pallas as pl
from jax.experimental.pallas import tpu as pltpu
from jax.experimental.pallas import tpu_sc as plsc

BATCH = 16384
DIM = 64
HIDDEN = 20
NC = 2
NS = 16
NW = NC * NS
B_PER_W = BATCH // NW
LANES = 16


def _sc_gather_mul(user_hbm, item_hbm, utab_hbm, itab_hbm, x_hbm,
                   uidx, iidx, urows, irows, sem_u, sem_i):
    wid = lax.axis_index("s") * NC + lax.axis_index("c")
    base = wid * B_PER_W
    pltpu.sync_copy(user_hbm.at[pl.ds(base, B_PER_W)], uidx)
    pltpu.sync_copy(item_hbm.at[pl.ds(base, B_PER_W)], iidx)

    def fire(g, carry):
        uv = uidx[pl.ds(g * LANES, LANES)] * DIM
        iv = iidx[pl.ds(g * LANES, LANES)] * DIM
        for k in range(LANES):
            j = (g * LANES + k) * DIM
            u_off = pl.multiple_of(uv[k], DIM)
            i_off = pl.multiple_of(iv[k], DIM)
            pltpu.async_copy(utab_hbm.at[pl.ds(u_off, DIM)],
                             urows.at[pl.ds(j, DIM)], sem_u)
            pltpu.async_copy(itab_hbm.at[pl.ds(i_off, DIM)],
                             irows.at[pl.ds(j, DIM)], sem_i)
        return carry

    lax.fori_loop(0, B_PER_W // LANES, fire, 0)
    pltpu.make_async_copy(utab_hbm.at[pl.ds(0, B_PER_W * DIM)], urows,
                          sem_u).wait()
    pltpu.make_async_copy(itab_hbm.at[pl.ds(0, B_PER_W * DIM)], irows,
                          sem_i).wait()

    def vec_body(v, carry):
        sl = pl.ds(v * LANES, LANES)
        urows[sl] = urows[sl] * irows[sl]
        return carry

    lax.fori_loop(0, B_PER_W * DIM // LANES, vec_body, 0)
    pltpu.sync_copy(urows, x_hbm.at[pl.ds(base * DIM, B_PER_W * DIM)])


@jax.jit
def _sc_stage(user, item, utab_flat, itab_flat):
    mesh = plsc.VectorSubcoreMesh(core_axis_name="c", subcore_axis_name="s")
    fn = pl.kernel(
        _sc_gather_mul,
        out_type=jax.ShapeDtypeStruct((BATCH * DIM,), jnp.float32),
        mesh=mesh,
        scratch_types=[
            pltpu.VMEM((B_PER_W,), jnp.int32),
            pltpu.VMEM((B_PER_W,), jnp.int32),
            pltpu.VMEM((B_PER_W * DIM,), jnp.float32),
            pltpu.VMEM((B_PER_W * DIM,), jnp.float32),
            pltpu.SemaphoreType.DMA,
            pltpu.SemaphoreType.DMA,
        ],
    )
    return fn(user, item, utab_flat, itab_flat)


def _tc_mlp_body(x_ref, w1t_ref, b1_ref, w2t_ref, b2_ref, o_ref):
    x = x_ref[...]
    h = jnp.dot(x, w1t_ref[...], preferred_element_type=jnp.float32)
    h = jnp.maximum(h + b1_ref[...], 0.0)
    z = jnp.dot(h, w2t_ref[...], preferred_element_type=jnp.float32)
    z = z + b2_ref[0, 0]
    o_ref[...] = 1.0 / (1.0 + jnp.exp(-z[:, 0]))


def _tc_mlp(x, W1T, b1r, W2T, b2r):
    blk = 2048
    grid = (BATCH // blk,)
    return pl.pallas_call(
        _tc_mlp_body,
        grid=grid,
        in_specs=[
            pl.BlockSpec((blk, DIM), lambda i: (i, 0)),
            pl.BlockSpec((DIM, HIDDEN), lambda i: (0, 0)),
            pl.BlockSpec((1, HIDDEN), lambda i: (0, 0)),
            pl.BlockSpec((HIDDEN, 1), lambda i: (0, 0)),
            pl.BlockSpec((1, 1), lambda i: (0, 0), memory_space=pltpu.SMEM),
        ],
        out_specs=pl.BlockSpec((blk,), lambda i: (i,)),
        out_shape=jax.ShapeDtypeStruct((BATCH,), jnp.float32),
    )(x, W1T, b1r, W2T, b2r)


def kernel(user, item, user_table, item_table, W1, b1, W2, b2):
    user = user.astype(jnp.int32)
    item = item.astype(jnp.int32)
    x = _sc_stage(user, item, user_table.reshape(-1), item_table.reshape(-1))
    x = x.reshape(BATCH, DIM)
    W1T = W1.T
    b1r = b1.reshape(1, HIDDEN)
    W2T = W2.T
    b2r = b2.reshape(1, 1)
    return _tc_mlp(x, W1T, b1r, W2T, b2r)


# speedup vs baseline: 1.5828x; 1.5828x over previous
import functools

import jax
import jax.numpy as jnp
from jax import lax
from jax.experimental import pallas as pl
from jax.experimental.pallas import tpu as pltpu
from jax.experimental.pallas import tpu_sc as plsc

BATCH = 16384
DIM = 64
HIDDEN = 20
NC = 2
NS = 16
NW = NC * NS
B_PER_W = BATCH // NW
CHUNK = 256
LANES = 16


def _sc_gather_mul(user_hbm, item_hbm, utab_hbm, itab_hbm, x_hbm,
                   uidx, iidx, urows, irows, sem_u, sem_i):
    wid = lax.axis_index("s") * NC + lax.axis_index("c")
    base = wid * B_PER_W
    pltpu.sync_copy(user_hbm.at[pl.ds(base, B_PER_W)], uidx)
    pltpu.sync_copy(item_hbm.at[pl.ds(base, B_PER_W)], iidx)

    for ch in range(B_PER_W // CHUNK):
        off = ch * CHUNK

        def fire(g, carry):
            uv = uidx[pl.ds(off + g * LANES, LANES)]
            iv = iidx[pl.ds(off + g * LANES, LANES)]
            for k in range(LANES):
                j = g * LANES + k
                pltpu.async_copy(utab_hbm.at[pl.ds(uv[k], 1)],
                                 urows.at[pl.ds(j, 1)], sem_u)
                pltpu.async_copy(itab_hbm.at[pl.ds(iv[k], 1)],
                                 irows.at[pl.ds(j, 1)], sem_i)
            return carry

        lax.fori_loop(0, CHUNK // LANES, fire, 0)
        pltpu.make_async_copy(utab_hbm.at[pl.ds(0, CHUNK)], urows, sem_u).wait()
        pltpu.make_async_copy(itab_hbm.at[pl.ds(0, CHUNK)], irows, sem_i).wait()

        def row_body(r, carry):
            for c in range(DIM // LANES):
                sl = pl.ds(c * LANES, LANES)
                urows[r, sl] = urows[r, sl] * irows[r, sl]
            return carry

        lax.fori_loop(0, CHUNK, row_body, 0)
        pltpu.sync_copy(urows, x_hbm.at[pl.ds(base + off, CHUNK)])


@jax.jit
def _sc_stage(user, item, user_table, item_table):
    mesh = plsc.VectorSubcoreMesh(core_axis_name="c", subcore_axis_name="s")
    fn = pl.kernel(
        _sc_gather_mul,
        out_type=jax.ShapeDtypeStruct((BATCH, DIM), jnp.float32),
        mesh=mesh,
        scratch_types=[
            pltpu.VMEM((B_PER_W,), jnp.int32),
            pltpu.VMEM((B_PER_W,), jnp.int32),
            pltpu.VMEM((CHUNK, DIM), jnp.float32),
            pltpu.VMEM((CHUNK, DIM), jnp.float32),
            pltpu.SemaphoreType.DMA,
            pltpu.SemaphoreType.DMA,
        ],
        compiler_params=pltpu.CompilerParams(needs_layout_passes=False),
    )
    return fn(user, item, user_table, item_table)


def _tc_mlp_body(x_ref, w1t_ref, b1_ref, w2t_ref, b2_ref, o_ref):
    x = x_ref[...]
    h = jnp.dot(x, w1t_ref[...], preferred_element_type=jnp.float32)
    h = jnp.maximum(h + b1_ref[...], 0.0)
    z = jnp.dot(h, w2t_ref[...], preferred_element_type=jnp.float32)
    z = z + b2_ref[0, 0]
    o_ref[...] = 1.0 / (1.0 + jnp.exp(-z[:, 0]))


def _tc_mlp(x, W1T, b1r, W2T, b2r):
    blk = 2048
    grid = (BATCH // blk,)
    return pl.pallas_call(
        _tc_mlp_body,
        grid=grid,
        in_specs=[
            pl.BlockSpec((blk, DIM), lambda i: (i, 0)),
            pl.BlockSpec((DIM, HIDDEN), lambda i: (0, 0)),
            pl.BlockSpec((1, HIDDEN), lambda i: (0, 0)),
            pl.BlockSpec((HIDDEN, 1), lambda i: (0, 0)),
            pl.BlockSpec((1, 1), lambda i: (0, 0), memory_space=pltpu.SMEM),
        ],
        out_specs=pl.BlockSpec((blk,), lambda i: (i,)),
        out_shape=jax.ShapeDtypeStruct((BATCH,), jnp.float32),
    )(x, W1T, b1r, W2T, b2r)


def kernel(user, item, user_table, item_table, W1, b1, W2, b2):
    user = user.astype(jnp.int32)
    item = item.astype(jnp.int32)
    x = _sc_stage(user, item, user_table, item_table)
    W1T = W1.T
    b1r = b1.reshape(1, HIDDEN)
    W2T = W2.T
    b2r = b2.reshape(1, 1)
    return _tc_mlp(x, W1T, b1r, W2T, b2r)


# speedup vs baseline: 2.1184x; 1.3384x over previous
import jax
import jax.numpy as jnp
from jax import lax
from jax.experimental import pallas as pl
from jax.experimental.pallas import tpu as pltpu
from jax.experimental.pallas import tpu_sc as plsc

BATCH = 16384
DIM = 64
HIDDEN = 20
NC = 2
NS = 16
NW = NC * NS
LANES = 16
V = 1000000
CH = 512
NCHUNK = 1953
TAIL = NCHUNK * CH
SLOTS = 64


def _sc_sweep(user_hbm, item_hbm, utabT_hbm, itabT_hbm, uo_hbm, vo_hbm,
              idxall, hpos, chunk, staging, mb, semo):
    wid = lax.axis_index("s") * NC + lax.axis_index("c")
    start = 61 * wid + jnp.minimum(wid, 1)
    nch = 61 + jnp.where(wid < 1, 1, 0)
    lo = CH * start
    hi = jnp.minimum(lo + CH * nch, TAIL)
    iota = lax.iota(jnp.int32, LANES)

    def phase(idx_src_hbm, tab_hbm, out_hbm):
        pltpu.sync_copy(idx_src_hbm, idxall)

        def scan_g(g, cnt):
            v = idxall[pl.ds(g * LANES, LANES)]
            m = (v >= lo) & (v < hi)
            key = jnp.where(m, 0, 1).astype(jnp.int32)
            _, pv = plsc.sort_key_val(key, iota + g * LANES)
            hpos[pl.ds(cnt, LANES)] = pv
            return cnt + plsc.all_reduce_population_count(m)[0]

        cnt = lax.fori_loop(0, BATCH // LANES, scan_g, 0)
        hpos[pl.ds(cnt, LANES)] = jnp.zeros((LANES,), jnp.int32)
        n_hg = (cnt + LANES - 1) // LANES

        def do_chunk(k, carry):
            ch_lo = pl.multiple_of(lo + k * CH, 128)
            pltpu.sync_copy(tab_hbm.at[:, pl.ds(ch_lo, CH)], chunk)

            def group(hg, mglob):
                hp = hpos[pl.ds(hg * LANES, LANES)]
                hv = plsc.load_gather(idxall, [hp])
                m2 = (hv >= ch_lo) & (hv < ch_lo + CH)
                packed = hp * CH + (hv - ch_lo)
                key = jnp.where(m2, 0, 1).astype(jnp.int32)
                _, pk = plsc.sort_key_val(key, packed)
                mb[pl.ds(0, LANES)] = pk
                pcnt = plsc.all_reduce_population_count(m2)[0]

                def match(h, mg):
                    q = mb[pl.ds(h, LANES)][0]
                    l = lax.rem(q, CH)
                    p = lax.div(q, CH)
                    slot = mg & (SLOTS - 1)

                    @pl.when(mg >= SLOTS)
                    def _():
                        pltpu.make_async_copy(
                            out_hbm.at[pl.ds(0, DIM)],
                            staging.at[pl.ds(0, DIM)], semo).wait()

                    cols = jnp.full((LANES,), l, jnp.int32)
                    for gg in range(DIM // LANES):
                        vals = plsc.load_gather(
                            chunk, [iota + gg * LANES, cols])
                        staging[pl.ds(slot * DIM + gg * LANES, LANES)] = vals
                    po = pl.multiple_of(p * DIM, 8)
                    pltpu.async_copy(staging.at[pl.ds(slot * DIM, DIM)],
                                     out_hbm.at[pl.ds(po, DIM)], semo)
                    return mg + 1

                return lax.fori_loop(0, pcnt, match, mglob)

            mglob = lax.fori_loop(0, n_hg, group, 0)

            def drain(_, c):
                pltpu.make_async_copy(out_hbm.at[pl.ds(0, DIM)],
                                      staging.at[pl.ds(0, DIM)], semo).wait()
                return c

            lax.fori_loop(0, jnp.minimum(mglob, SLOTS), drain, 0)
            return carry

        lax.fori_loop(0, nch, do_chunk, 0)

    phase(user_hbm, utabT_hbm, uo_hbm)
    phase(item_hbm, itabT_hbm, vo_hbm)


@jax.jit
def _sc_stage(user, item, utabT, itabT):
    mesh = plsc.VectorSubcoreMesh(core_axis_name="c", subcore_axis_name="s")
    fn = pl.kernel(
        _sc_sweep,
        out_type=(jax.ShapeDtypeStruct((BATCH * DIM,), jnp.float32),
                  jax.ShapeDtypeStruct((BATCH * DIM,), jnp.float32)),
        mesh=mesh,
        scratch_types=[
            pltpu.VMEM((BATCH,), jnp.int32),
            pltpu.VMEM((BATCH + LANES,), jnp.int32),
            pltpu.VMEM((DIM, CH), jnp.float32),
            pltpu.VMEM((SLOTS * DIM,), jnp.float32),
            pltpu.VMEM((2 * LANES,), jnp.int32),
            pltpu.SemaphoreType.DMA,
        ],
        compiler_params=pltpu.CompilerParams(needs_layout_passes=False),
    )
    return fn(user, item, utabT, itabT)


def _tc_mlp_body(u_ref, v_ref, w1t_ref, b1_ref, w2t_ref, b2_ref, o_ref):
    x = u_ref[...] * v_ref[...]
    h = jnp.dot(x, w1t_ref[...], preferred_element_type=jnp.float32)
    h = jnp.maximum(h + b1_ref[...], 0.0)
    z = jnp.dot(h, w2t_ref[...], preferred_element_type=jnp.float32)
    z = z + b2_ref[0, 0]
    o_ref[...] = 1.0 / (1.0 + jnp.exp(-z[:, 0]))


def _tc_mlp(u2d, v2d, W1T, b1r, W2T, b2r):
    blk = 2048
    grid = (BATCH // blk,)
    return pl.pallas_call(
        _tc_mlp_body,
        grid=grid,
        in_specs=[
            pl.BlockSpec((blk, DIM), lambda i: (i, 0)),
            pl.BlockSpec((blk, DIM), lambda i: (i, 0)),
            pl.BlockSpec((DIM, HIDDEN), lambda i: (0, 0)),
            pl.BlockSpec((1, HIDDEN), lambda i: (0, 0)),
            pl.BlockSpec((HIDDEN, 1), lambda i: (0, 0)),
            pl.BlockSpec((1, 1), lambda i: (0, 0), memory_space=pltpu.SMEM),
        ],
        out_specs=pl.BlockSpec((blk,), lambda i: (i,)),
        out_shape=jax.ShapeDtypeStruct((BATCH,), jnp.float32),
    )(u2d, v2d, W1T, b1r, W2T, b2r)


def kernel(user, item, user_table, item_table, W1, b1, W2, b2):
    user = user.astype(jnp.int32)
    item = item.astype(jnp.int32)
    user_c = jnp.minimum(user, TAIL - 1)
    item_c = jnp.minimum(item, TAIL - 1)
    u_flat, v_flat = _sc_stage(user_c, item_c, user_table.T, item_table.T)
    u2d = u_flat.reshape(BATCH, DIM)
    v2d = v_flat.reshape(BATCH, DIM)
    tail_u = user_table[TAIL:, :]
    tail_v = item_table[TAIL:, :]
    um = user >= TAIL
    vm = item >= TAIL
    u2d = jnp.where(um[:, None], tail_u[jnp.maximum(user - TAIL, 0)], u2d)
    v2d = jnp.where(vm[:, None], tail_v[jnp.maximum(item - TAIL, 0)], v2d)
    W1T = W1.T
    b1r = b1.reshape(1, HIDDEN)
    W2T = W2.T
    b2r = b2.reshape(1, 1)
    return _tc_mlp(u2d, v2d, W1T, b1r, W2T, b2r)
